# baseline (speedup 1.0000x reference)
import jax
import jax.numpy as jnp
from jax.experimental import pallas as pl
from jax.experimental.pallas import tpu as pltpu

_LOG2E = 1.4426950408889634
_LN2 = 0.6931471805599453


def _bce_block(x, y):
    l = jnp.log2(1.0 + jnp.exp2(jnp.abs(x) * -_LOG2E))
    return jnp.maximum(x, 0.0) - x * y + _LN2 * l


def _reduce_loss(x2, t2, nb, tb, width, inv_b):
    def body(x_ref, t_ref, o_ref):
        j = pl.program_id(0)

        @pl.when(j == 0)
        def _():
            o_ref[...] = jnp.zeros_like(o_ref)

        o_ref[...] += jnp.sum(_bce_block(x_ref[...], t_ref[...]),
                              axis=(0, 1), keepdims=True)

        @pl.when(j == pl.num_programs(0) - 1)
        def _():
            o_ref[...] *= jnp.float32(inv_b)

    out = pl.pallas_call(
        body,
        out_shape=jax.ShapeDtypeStruct((1, 1), jnp.float32),
        grid_spec=pltpu.PrefetchScalarGridSpec(
            num_scalar_prefetch=0,
            grid=(nb,),
            in_specs=[
                pl.BlockSpec((tb, width), lambda j: (j, 0)),
                pl.BlockSpec((1, width), lambda j: (0, 0)),
            ],
            out_specs=pl.BlockSpec((1, 1), lambda j: (0, 0)),
        ),
        compiler_params=pltpu.CompilerParams(
            dimension_semantics=("arbitrary",)),
    )(x2, t2)
    return out[0, 0]


def kernel(img, msg_logits, target_msg):
    del img
    B, bits = msg_logits.shape
    inv_b = 1.0 / float(B)

    lanes = 128
    while lanes % bits:
        lanes += 128
    rep = lanes // bits
    total = B * bits

    if total % lanes == 0:
        rows = total // lanes
        tb = next((t for t in (4096, 2048, 1024, 512, 256, 128, 64, 32,
                               16, 8, 4, 2, 1) if rows % t == 0))
        x2 = msg_logits.reshape(rows, lanes)
        t2 = jnp.tile(target_msg, rep).reshape(1, lanes)
        return _reduce_loss(x2, t2, rows // tb, tb, lanes, inv_b)

    tb = B if B <= 4096 else 4096
    nb = pl.cdiv(B, tb)
    ragged = (B % tb) != 0

    def body(x_ref, t_ref, o_ref):
        j = pl.program_id(0)

        @pl.when(j == 0)
        def _():
            o_ref[...] = jnp.zeros_like(o_ref)

        per = _bce_block(x_ref[...], t_ref[...])
        if ragged:
            row = jax.lax.broadcasted_iota(jnp.int32, per.shape, 0) + j * tb
            per = jnp.where(row < B, per, 0.0)
        o_ref[...] += jnp.sum(per, axis=(0, 1), keepdims=True)

        @pl.when(j == pl.num_programs(0) - 1)
        def _():
            o_ref[...] *= jnp.float32(inv_b)

    out = pl.pallas_call(
        body,
        out_shape=jax.ShapeDtypeStruct((1, 1), jnp.float32),
        grid_spec=pltpu.PrefetchScalarGridSpec(
            num_scalar_prefetch=0,
            grid=(nb,),
            in_specs=[
                pl.BlockSpec((tb, bits), lambda j: (j, 0)),
                pl.BlockSpec((1, bits), lambda j: (0, 0)),
            ],
            out_specs=pl.BlockSpec((1, 1), lambda j: (0, 0)),
        ),
        compiler_params=pltpu.CompilerParams(
            dimension_semantics=("arbitrary",)),
    )(msg_logits, target_msg.reshape(1, bits))
    return out[0, 0]
